# baseline (speedup 1.0000x reference)
import functools

import jax
import jax.numpy as jnp
from jax import lax
from jax.experimental import pallas as pl
from jax.experimental.pallas import tpu as pltpu
from jax.experimental.pallas import tpu_sc as plsc

_ID_DIM = 64
_CAT_DIM = 64
_FINAL = _ID_DIM + _CAT_DIM
_B = 4096
_L = 200
_VOCAB = 1000000
_CAT_VOCAB = 100000

_NC = 2
_NS = 16
_NW = _NC * _NS
_COLS_W = _B // _NW
_NBUF = 4
_STEPS = _L // _NBUF
_TBLK = 512


def _transpose_body(src_ref, dst_ref):
    a = src_ref[...]
    eye = jnp.eye(64, dtype=jnp.float32)
    t = lax.dot_general(a, eye, (((0,), (0,)), ((), ())),
                        precision=lax.Precision.HIGHEST,
                        preferred_element_type=jnp.float32)
    dst_ref[...] = jnp.concatenate([t, jnp.zeros_like(t)], axis=1)


def _transpose_table(tab_t, v):
    grid = (v + _TBLK - 1) // _TBLK
    return pl.pallas_call(
        _transpose_body,
        grid=(grid,),
        in_specs=[pl.BlockSpec((64, _TBLK), lambda i: (0, i))],
        out_specs=pl.BlockSpec((_TBLK, _FINAL), lambda i: (i, 0)),
        out_shape=jax.ShapeDtypeStruct((v, _FINAL), jnp.float32),
    )(tab_t)


def _sc_body(sample_t_hbm, id2_hbm, cat2_hbm, cmap_hbm, out_hbm,
             idx_all, tok2_v, cidx_v, idr_v, catr_v,
             sem_cmap, sem_rows, sem_out):
    wid = lax.axis_index("s") * _NC + lax.axis_index("c")
    col0 = wid * _COLS_W

    pltpu.sync_copy(sample_t_hbm.at[:, pl.ds(col0, _COLS_W)], idx_all)

    def out_slices(l):
        return (out_hbm.at[pl.ds(col0, _COLS_W), l, pl.ds(0, _ID_DIM)],
                out_hbm.at[pl.ds(col0, _COLS_W), l, pl.ds(_ID_DIM, _CAT_DIM)])

    def front(i, l, b):
        @pl.when(i >= 1)
        def _():
            oid, ocat = out_slices(l - _NBUF)
            pltpu.make_async_copy(oid, idr_v[b], sem_out[b]).wait()
            pltpu.make_async_copy(ocat, catr_v[b], sem_out[b]).wait()
        pltpu.async_copy(cmap_hbm.at[idx_all.at[l, :]],
                         cidx_v[b], sem_cmap[b])
        for t in range(_COLS_W // 16):
            sl = pl.ds(t * 16, 16)
            tok = idx_all[l, sl]
            tok2_v[b][sl] = tok + tok
        pltpu.async_copy(id2_hbm.at[tok2_v[b]], idr_v[b], sem_rows[b])

    def mid(l, b):
        pltpu.make_async_copy(cmap_hbm.at[idx_all.at[l, :]],
                              cidx_v[b], sem_cmap[b]).wait()
        for t in range(_COLS_W // 16):
            sl = pl.ds(t * 16, 16)
            tok = idx_all[l, sl]
            cid = cidx_v[b][sl]
            cidx_v[b][sl] = jnp.where(tok == 0, jnp.zeros_like(cid),
                                      cid + cid)
        pltpu.async_copy(cat2_hbm.at[cidx_v[b]], catr_v[b], sem_rows[b])

    def back(l, b):
        pltpu.make_async_copy(id2_hbm.at[tok2_v[b]],
                              idr_v[b], sem_rows[b]).wait()
        pltpu.make_async_copy(cat2_hbm.at[cidx_v[b]],
                              catr_v[b], sem_rows[b]).wait()
        oid, ocat = out_slices(l)
        pltpu.async_copy(idr_v[b], oid, sem_out[b])
        pltpu.async_copy(catr_v[b], ocat, sem_out[b])

    def step(i, carry):
        for b in range(_NBUF):
            front(i, i * _NBUF + b, b)
        for b in range(_NBUF):
            mid(i * _NBUF + b, b)
        for b in range(_NBUF):
            back(i * _NBUF + b, b)
        return carry

    lax.fori_loop(0, _STEPS, step, 0)

    for b in range(_NBUF):
        oid, ocat = out_slices(_L - _NBUF + b)
        pltpu.make_async_copy(oid, idr_v[b], sem_out[b]).wait()
        pltpu.make_async_copy(ocat, catr_v[b], sem_out[b]).wait()


@jax.jit
def kernel(sample, item_id_table, category_table, category_map):
    id_packed = _transpose_table(item_id_table.T, _VOCAB)
    cat_packed = _transpose_table(category_table.T, _CAT_VOCAB)
    id2 = id_packed.reshape(2 * _VOCAB, _ID_DIM)
    cat2 = cat_packed.reshape(2 * _CAT_VOCAB, _CAT_DIM)

    mesh = plsc.VectorSubcoreMesh(core_axis_name="c", subcore_axis_name="s")
    f = functools.partial(
        pl.kernel,
        out_type=jax.ShapeDtypeStruct((_B, _L, _FINAL), jnp.float32),
        mesh=mesh,
        compiler_params=pltpu.CompilerParams(use_tc_tiling_on_sc=False),
        scratch_types=[
            pltpu.VMEM((_L, _COLS_W), jnp.int32),
            [pltpu.VMEM((_COLS_W,), jnp.int32)] * _NBUF,
            [pltpu.VMEM((_COLS_W,), jnp.int32)] * _NBUF,
            [pltpu.VMEM((_COLS_W, _ID_DIM), jnp.float32)] * _NBUF,
            [pltpu.VMEM((_COLS_W, _CAT_DIM), jnp.float32)] * _NBUF,
            [pltpu.SemaphoreType.DMA] * _NBUF,
            [pltpu.SemaphoreType.DMA] * _NBUF,
            [pltpu.SemaphoreType.DMA] * _NBUF,
        ],
    )(_sc_body)
    return f(sample.T, id2, cat2, category_map)


# speedup vs baseline: 1.0812x; 1.0812x over previous
import functools

import jax
import jax.numpy as jnp
from jax import lax
from jax.experimental import pallas as pl
from jax.experimental.pallas import tpu as pltpu
from jax.experimental.pallas import tpu_sc as plsc

_ID_DIM = 64
_CAT_DIM = 64
_FINAL = _ID_DIM + _CAT_DIM
_B = 4096
_L = 200
_VOCAB = 1000000
_CAT_VOCAB = 100000

_NC = 2
_NS = 16
_NW = _NC * _NS
_COLS_W = _B // _NW
_NBUF = 4
_STEPS = _L // _NBUF
_TBLK = 512


def _transpose_body(src_ref, dst_ref):
    a = src_ref[...]
    eye = jnp.eye(64, dtype=jnp.float32)
    t = lax.dot_general(a, eye, (((0,), (0,)), ((), ())),
                        preferred_element_type=jnp.float32)
    dst_ref[...] = jnp.concatenate([t, jnp.zeros_like(t)], axis=1)


def _transpose_table(tab_t, v):
    grid = (v + _TBLK - 1) // _TBLK
    return pl.pallas_call(
        _transpose_body,
        grid=(grid,),
        in_specs=[pl.BlockSpec((64, _TBLK), lambda i: (0, i))],
        out_specs=pl.BlockSpec((_TBLK, _FINAL), lambda i: (i, 0)),
        out_shape=jax.ShapeDtypeStruct((v, _FINAL), jnp.float32),
    )(tab_t)


def _sc_body(sample_t_hbm, id2_hbm, cat2_hbm, cmap_hbm, out_hbm,
             idx_all, tok2_v, cidx_v, idr_v, catr_v,
             sem_cmap, sem_rows, sem_out):
    wid = lax.axis_index("s") * _NC + lax.axis_index("c")
    col0 = wid * _COLS_W

    pltpu.sync_copy(sample_t_hbm.at[:, pl.ds(col0, _COLS_W)], idx_all)

    def out_slices(l):
        return (out_hbm.at[pl.ds(col0, _COLS_W), l, pl.ds(0, _ID_DIM)],
                out_hbm.at[pl.ds(col0, _COLS_W), l, pl.ds(_ID_DIM, _CAT_DIM)])

    def front(i, l, b):
        @pl.when(i >= 1)
        def _():
            oid, ocat = out_slices(l - _NBUF)
            pltpu.make_async_copy(oid, idr_v[b], sem_out[b]).wait()
            pltpu.make_async_copy(ocat, catr_v[b], sem_out[b]).wait()
        pltpu.async_copy(cmap_hbm.at[idx_all.at[l, :]],
                         cidx_v[b], sem_cmap[b])
        for t in range(_COLS_W // 16):
            sl = pl.ds(t * 16, 16)
            tok = idx_all[l, sl]
            tok2_v[b][sl] = tok + tok
        pltpu.async_copy(id2_hbm.at[tok2_v[b]], idr_v[b], sem_rows[b])

    def mid(l, b):
        pltpu.make_async_copy(cmap_hbm.at[idx_all.at[l, :]],
                              cidx_v[b], sem_cmap[b]).wait()
        for t in range(_COLS_W // 16):
            sl = pl.ds(t * 16, 16)
            tok = idx_all[l, sl]
            cid = cidx_v[b][sl]
            cidx_v[b][sl] = jnp.where(tok == 0, jnp.zeros_like(cid),
                                      cid + cid)
        pltpu.async_copy(cat2_hbm.at[cidx_v[b]], catr_v[b], sem_rows[b])

    def back(l, b):
        pltpu.make_async_copy(id2_hbm.at[tok2_v[b]],
                              idr_v[b], sem_rows[b]).wait()
        pltpu.make_async_copy(cat2_hbm.at[cidx_v[b]],
                              catr_v[b], sem_rows[b]).wait()
        oid, ocat = out_slices(l)
        pltpu.async_copy(idr_v[b], oid, sem_out[b])
        pltpu.async_copy(catr_v[b], ocat, sem_out[b])

    def step(i, carry):
        for b in range(_NBUF):
            front(i, i * _NBUF + b, b)
        for b in range(_NBUF):
            mid(i * _NBUF + b, b)
        for b in range(_NBUF):
            back(i * _NBUF + b, b)
        return carry

    lax.fori_loop(0, _STEPS, step, 0)

    for b in range(_NBUF):
        oid, ocat = out_slices(_L - _NBUF + b)
        pltpu.make_async_copy(oid, idr_v[b], sem_out[b]).wait()
        pltpu.make_async_copy(ocat, catr_v[b], sem_out[b]).wait()


@jax.jit
def kernel(sample, item_id_table, category_table, category_map):
    id_packed = _transpose_table(item_id_table.T, _VOCAB)
    cat_packed = _transpose_table(category_table.T, _CAT_VOCAB)
    id2 = id_packed.reshape(2 * _VOCAB, _ID_DIM)
    cat2 = cat_packed.reshape(2 * _CAT_VOCAB, _CAT_DIM)

    mesh = plsc.VectorSubcoreMesh(core_axis_name="c", subcore_axis_name="s")
    f = functools.partial(
        pl.kernel,
        out_type=jax.ShapeDtypeStruct((_B, _L, _FINAL), jnp.float32),
        mesh=mesh,
        compiler_params=pltpu.CompilerParams(use_tc_tiling_on_sc=False),
        scratch_types=[
            pltpu.VMEM((_L, _COLS_W), jnp.int32),
            [pltpu.VMEM((_COLS_W,), jnp.int32)] * _NBUF,
            [pltpu.VMEM((_COLS_W,), jnp.int32)] * _NBUF,
            [pltpu.VMEM((_COLS_W, _ID_DIM), jnp.float32)] * _NBUF,
            [pltpu.VMEM((_COLS_W, _CAT_DIM), jnp.float32)] * _NBUF,
            [pltpu.SemaphoreType.DMA] * _NBUF,
            [pltpu.SemaphoreType.DMA] * _NBUF,
            [pltpu.SemaphoreType.DMA] * _NBUF,
        ],
    )(_sc_body)
    return f(sample.T, id2, cat2, category_map)


# speedup vs baseline: 1.9662x; 1.8186x over previous
import functools

import jax
import jax.numpy as jnp
from jax import lax
from jax.experimental import pallas as pl
from jax.experimental.pallas import tpu as pltpu
from jax.experimental.pallas import tpu_sc as plsc

_ID_DIM = 64
_CAT_DIM = 64
_FINAL = _ID_DIM + _CAT_DIM
_B = 4096
_L = 200

_NC = 2
_NS = 16
_NW = _NC * _NS
_COLS_W = _B // _NW
_NBUF = 6
_REM = _L % _NBUF
_STEPS = _L // _NBUF


def _sc_body(sample_t_hbm, id_tab_hbm, cat_tab_hbm, cmap_hbm, out_hbm,
             idx_all, cidx_v, idr_v, catr_v, sem_cmap, sem_rows, sem_out):
    wid = lax.axis_index("s") * _NC + lax.axis_index("c")
    col0 = wid * _COLS_W

    pltpu.sync_copy(sample_t_hbm.at[:, pl.ds(col0, _COLS_W)], idx_all)

    def out_slices(l):
        return (out_hbm.at[pl.ds(col0, _COLS_W), l, pl.ds(0, _ID_DIM)],
                out_hbm.at[pl.ds(col0, _COLS_W), l, pl.ds(_ID_DIM, _CAT_DIM)])

    def front(l, b, first):
        if first is None:
            _drain_out(l - _NBUF, b)
        else:
            @pl.when(first)
            def _():
                _drain_out(l - _NBUF, b)
        pltpu.async_copy(cmap_hbm.at[idx_all.at[l, :]],
                         cidx_v[b], sem_cmap[b])
        pltpu.async_copy(id_tab_hbm.at[idx_all.at[l, :]],
                         idr_v[b], sem_rows[b])

    def _drain_out(l, b):
        oid, ocat = out_slices(l)
        pltpu.make_async_copy(oid, idr_v[b], sem_out[b]).wait()
        pltpu.make_async_copy(ocat, catr_v[b], sem_out[b]).wait()

    def mid(l, b):
        pltpu.make_async_copy(cmap_hbm.at[idx_all.at[l, :]],
                              cidx_v[b], sem_cmap[b]).wait()
        for t in range(_COLS_W // 16):
            sl = pl.ds(t * 16, 16)
            tok = idx_all[l, sl]
            cid = cidx_v[b][sl]
            cidx_v[b][sl] = jnp.where(tok == 0, jnp.zeros_like(cid), cid)
        pltpu.async_copy(cat_tab_hbm.at[cidx_v[b]], catr_v[b], sem_rows[b])

    def back(l, b):
        pltpu.make_async_copy(id_tab_hbm.at[idx_all.at[l, :]],
                              idr_v[b], sem_rows[b]).wait()
        pltpu.make_async_copy(cat_tab_hbm.at[cidx_v[b]],
                              catr_v[b], sem_rows[b]).wait()
        oid, ocat = out_slices(l)
        pltpu.async_copy(idr_v[b], oid, sem_out[b])
        pltpu.async_copy(catr_v[b], ocat, sem_out[b])

    def step(i, carry):
        for b in range(_NBUF):
            front(i * _NBUF + b, b, i >= 1)
        for b in range(_NBUF):
            mid(i * _NBUF + b, b)
        for b in range(_NBUF):
            back(i * _NBUF + b, b)
        return carry

    lax.fori_loop(0, _STEPS, step, 0)

    base = _STEPS * _NBUF
    for b in range(_REM):
        front(base + b, b, None)
    for b in range(_REM):
        mid(base + b, b)
    for b in range(_REM):
        back(base + b, b)

    for b in range(_REM, _NBUF):
        _drain_out(base - _NBUF + b, b)
    for b in range(_REM):
        _drain_out(base + b, b)


@jax.jit
def kernel(sample, item_id_table, category_table, category_map):
    mesh = plsc.VectorSubcoreMesh(core_axis_name="c", subcore_axis_name="s")
    f = functools.partial(
        pl.kernel,
        out_type=jax.ShapeDtypeStruct((_B, _L, _FINAL), jnp.float32),
        mesh=mesh,
        compiler_params=pltpu.CompilerParams(use_tc_tiling_on_sc=False),
        scratch_types=[
            pltpu.VMEM((_L, _COLS_W), jnp.int32),
            [pltpu.VMEM((_COLS_W,), jnp.int32)] * _NBUF,
            [pltpu.VMEM((_COLS_W, _ID_DIM), jnp.float32)] * _NBUF,
            [pltpu.VMEM((_COLS_W, _CAT_DIM), jnp.float32)] * _NBUF,
            [pltpu.SemaphoreType.DMA] * _NBUF,
            [pltpu.SemaphoreType.DMA] * _NBUF,
            [pltpu.SemaphoreType.DMA] * _NBUF,
        ],
    )(_sc_body)
    return f(sample.T, item_id_table, category_table, category_map)


# speedup vs baseline: 1.9688x; 1.0013x over previous
import functools

import jax
import jax.numpy as jnp
from jax import lax
from jax.experimental import pallas as pl
from jax.experimental.pallas import tpu as pltpu
from jax.experimental.pallas import tpu_sc as plsc

_ID_DIM = 64
_CAT_DIM = 64
_FINAL = _ID_DIM + _CAT_DIM
_B = 4096
_L = 200

_NC = 2
_NS = 16
_NW = _NC * _NS
_COLS_W = _B // _NW
_NBUF = 6
_REM = _L % _NBUF
_STEPS = _L // _NBUF


def _sc_body(sample_t_hbm, id_tab_hbm, cat_tab_hbm, cmap_hbm, out_hbm,
             idx_all, cidx_v, idr_v, catr_v, sem_cmap, sem_rows, sem_out):
    wid = lax.axis_index("s") * _NC + lax.axis_index("c")
    col0 = wid * _COLS_W

    pltpu.sync_copy(sample_t_hbm.at[:, pl.ds(col0, _COLS_W)], idx_all)

    def out_slices(l):
        return (out_hbm.at[pl.ds(col0, _COLS_W), l, pl.ds(0, _ID_DIM)],
                out_hbm.at[pl.ds(col0, _COLS_W), l, pl.ds(_ID_DIM, _CAT_DIM)])

    def front(l, b, first):
        if first is None:
            _drain_out(l - _NBUF, b)
        else:
            @pl.when(first)
            def _():
                _drain_out(l - _NBUF, b)
        pltpu.async_copy(cmap_hbm.at[idx_all.at[l, :]],
                         cidx_v[b], sem_cmap[b])
        pltpu.async_copy(id_tab_hbm.at[idx_all.at[l, :]],
                         idr_v[b], sem_rows[b])

    def _drain_out(l, b):
        oid, ocat = out_slices(l)
        pltpu.make_async_copy(oid, idr_v[b], sem_out[b]).wait()
        pltpu.make_async_copy(ocat, catr_v[b], sem_out[b]).wait()

    def mid(l, b):
        pltpu.make_async_copy(cmap_hbm.at[idx_all.at[l, :]],
                              cidx_v[b], sem_cmap[b]).wait()
        for t in range(_COLS_W // 16):
            sl = pl.ds(t * 16, 16)
            tok = idx_all[l, sl]
            cid = cidx_v[b][sl]
            cidx_v[b][sl] = jnp.where(tok == 0, jnp.zeros_like(cid), cid)
        pltpu.async_copy(cat_tab_hbm.at[cidx_v[b]], catr_v[b], sem_rows[b])

    def back(l, b):
        pltpu.make_async_copy(id_tab_hbm.at[idx_all.at[l, :]],
                              idr_v[b], sem_rows[b]).wait()
        pltpu.make_async_copy(cat_tab_hbm.at[cidx_v[b]],
                              catr_v[b], sem_rows[b]).wait()
        oid, ocat = out_slices(l)
        pltpu.async_copy(idr_v[b], oid, sem_out[b])
        pltpu.async_copy(catr_v[b], ocat, sem_out[b])

    def step(i, carry):
        for b in range(_NBUF):
            front(i * _NBUF + b, b, i >= 1)
        for b in range(_NBUF):
            mid(i * _NBUF + b, b)
        for b in range(_NBUF):
            back(i * _NBUF + b, b)
        return carry

    lax.fori_loop(0, _STEPS, step, 0)

    base = _STEPS * _NBUF
    for b in range(_REM):
        front(base + b, b, None)
    for b in range(_REM):
        mid(base + b, b)
    for b in range(_REM):
        back(base + b, b)

    for b in range(_REM, _NBUF):
        _drain_out(base - _NBUF + b, b)
    for b in range(_REM):
        _drain_out(base + b, b)


@jax.jit
def kernel(sample, item_id_table, category_table, category_map):
    mesh = plsc.VectorSubcoreMesh(core_axis_name="c", subcore_axis_name="s")
    f = functools.partial(
        pl.kernel,
        out_type=jax.ShapeDtypeStruct((_B, _L, _FINAL), jnp.float32),
        mesh=mesh,
        compiler_params=pltpu.CompilerParams(use_tc_tiling_on_sc=False),
        scratch_types=[
            pltpu.VMEM((_L, _COLS_W), jnp.int32),
            [pltpu.VMEM((_COLS_W,), jnp.int32)] * _NBUF,
            [pltpu.VMEM((_COLS_W, _ID_DIM), jnp.float32)] * _NBUF,
            [pltpu.VMEM((_COLS_W, _CAT_DIM), jnp.float32)] * _NBUF,
            [pltpu.SemaphoreType.DMA] * _NBUF,
            [pltpu.SemaphoreType.DMA] * _NBUF,
            [pltpu.SemaphoreType.DMA] * _NBUF,
        ],
    )(_sc_body)
    return f(sample.T, item_id_table + 0.0, category_table + 0.0,
             category_map)


# speedup vs baseline: 2.6976x; 1.3702x over previous
import functools

import jax
import jax.numpy as jnp
from jax import lax
from jax.experimental import pallas as pl
from jax.experimental.pallas import tpu as pltpu
from jax.experimental.pallas import tpu_sc as plsc

_ID_DIM = 64
_CAT_DIM = 64
_FINAL = _ID_DIM + _CAT_DIM
_B = 4096
_L = 200

_NC = 2
_NS = 16
_NW = _NC * _NS
_COLS_W = _B // _NW
_NBUF = 6
_REM = _L % _NBUF
_STEPS = _L // _NBUF
_TBLK = 4096


def _transpose_body(src_ref, dst_ref):
    t = src_ref[...].T
    dst_ref[:, 0:64] = t
    dst_ref[:, 64:128] = jnp.zeros_like(t)


def _transpose_table(tab_t, v):
    grid = (v + _TBLK - 1) // _TBLK
    return pl.pallas_call(
        _transpose_body,
        grid=(grid,),
        in_specs=[pl.BlockSpec((64, _TBLK), lambda i: (0, i))],
        out_specs=pl.BlockSpec((_TBLK, _FINAL), lambda i: (i, 0)),
        out_shape=jax.ShapeDtypeStruct((v, _FINAL), jnp.float32),
    )(tab_t)


def _sc_body(sample_t_hbm, id_tab_hbm, cat_tab_hbm, cmap_hbm, out_hbm,
             idx_all, tok2_v, cidx_v, idr_v, catr_v, sem_cmap, sem_rows, sem_out):
    wid = lax.axis_index("s") * _NC + lax.axis_index("c")
    col0 = wid * _COLS_W

    pltpu.sync_copy(sample_t_hbm.at[:, pl.ds(col0, _COLS_W)], idx_all)

    def out_slices(l):
        return (out_hbm.at[pl.ds(col0, _COLS_W), l, pl.ds(0, _ID_DIM)],
                out_hbm.at[pl.ds(col0, _COLS_W), l, pl.ds(_ID_DIM, _CAT_DIM)])

    def front(l, b, first):
        if first is None:
            _drain_out(l - _NBUF, b)
        else:
            @pl.when(first)
            def _():
                _drain_out(l - _NBUF, b)
        pltpu.async_copy(cmap_hbm.at[idx_all.at[l, :]],
                         cidx_v[b], sem_cmap[b])
        for t in range(_COLS_W // 16):
            sl = pl.ds(t * 16, 16)
            tok = idx_all[l, sl]
            tok2_v[b][sl] = tok + tok
        pltpu.async_copy(id_tab_hbm.at[tok2_v[b]],
                         idr_v[b], sem_rows[b])

    def _drain_out(l, b):
        oid, ocat = out_slices(l)
        pltpu.make_async_copy(oid, idr_v[b], sem_out[b]).wait()
        pltpu.make_async_copy(ocat, catr_v[b], sem_out[b]).wait()

    def mid(l, b):
        pltpu.make_async_copy(cmap_hbm.at[idx_all.at[l, :]],
                              cidx_v[b], sem_cmap[b]).wait()
        for t in range(_COLS_W // 16):
            sl = pl.ds(t * 16, 16)
            tok = idx_all[l, sl]
            cid = cidx_v[b][sl]
            cidx_v[b][sl] = jnp.where(tok == 0, jnp.zeros_like(cid),
                                      cid + cid)
        pltpu.async_copy(cat_tab_hbm.at[cidx_v[b]], catr_v[b], sem_rows[b])

    def back(l, b):
        pltpu.make_async_copy(id_tab_hbm.at[tok2_v[b]],
                              idr_v[b], sem_rows[b]).wait()
        pltpu.make_async_copy(cat_tab_hbm.at[cidx_v[b]],
                              catr_v[b], sem_rows[b]).wait()
        oid, ocat = out_slices(l)
        pltpu.async_copy(idr_v[b], oid, sem_out[b])
        pltpu.async_copy(catr_v[b], ocat, sem_out[b])

    def step(i, carry):
        for b in range(_NBUF):
            front(i * _NBUF + b, b, i >= 1)
        for b in range(_NBUF):
            mid(i * _NBUF + b, b)
        for b in range(_NBUF):
            back(i * _NBUF + b, b)
        return carry

    lax.fori_loop(0, _STEPS, step, 0)

    base = _STEPS * _NBUF
    for b in range(_REM):
        front(base + b, b, None)
    for b in range(_REM):
        mid(base + b, b)
    for b in range(_REM):
        back(base + b, b)

    for b in range(_REM, _NBUF):
        _drain_out(base - _NBUF + b, b)
    for b in range(_REM):
        _drain_out(base + b, b)


@jax.jit
def kernel(sample, item_id_table, category_table, category_map):
    mesh = plsc.VectorSubcoreMesh(core_axis_name="c", subcore_axis_name="s")
    f = functools.partial(
        pl.kernel,
        out_type=jax.ShapeDtypeStruct((_B, _L, _FINAL), jnp.float32),
        mesh=mesh,
        compiler_params=pltpu.CompilerParams(use_tc_tiling_on_sc=False),
        scratch_types=[
            pltpu.VMEM((_L, _COLS_W), jnp.int32),
            [pltpu.VMEM((_COLS_W,), jnp.int32)] * _NBUF,
            [pltpu.VMEM((_COLS_W,), jnp.int32)] * _NBUF,
            [pltpu.VMEM((_COLS_W, _ID_DIM), jnp.float32)] * _NBUF,
            [pltpu.VMEM((_COLS_W, _CAT_DIM), jnp.float32)] * _NBUF,
            [pltpu.SemaphoreType.DMA] * _NBUF,
            [pltpu.SemaphoreType.DMA] * _NBUF,
            [pltpu.SemaphoreType.DMA] * _NBUF,
        ],
    )(_sc_body)
    id2 = _transpose_table(item_id_table.T, 1000000).reshape(2000000, _ID_DIM)
    cat2 = _transpose_table(category_table.T, 100000).reshape(200000, _CAT_DIM)
    return f(sample.T, id2, cat2, category_map)


# speedup vs baseline: 3.0569x; 1.1332x over previous
import functools

import jax
import jax.numpy as jnp
from jax import lax
from jax.experimental import pallas as pl
from jax.experimental.pallas import tpu as pltpu
from jax.experimental.pallas import tpu_sc as plsc

_ID_DIM = 64
_CAT_DIM = 64
_FINAL = _ID_DIM + _CAT_DIM
_B = 4096
_L = 200

_NC = 2
_NS = 16
_NW = _NC * _NS
_COLS_W = _B // _NW
_NBUF = 6
_REM = _L % _NBUF
_STEPS = _L // _NBUF
_TBLK = 8192


def _transpose_body(src_ref, dst_ref):
    t = src_ref[...].T
    dst_ref[:, 0:64] = t
    dst_ref[:, 64:128] = jnp.zeros_like(t)


def _transpose_table(tab_t, v):
    grid = (v + _TBLK - 1) // _TBLK
    return pl.pallas_call(
        _transpose_body,
        grid=(grid,),
        in_specs=[pl.BlockSpec((64, _TBLK), lambda i: (0, i))],
        out_specs=pl.BlockSpec((_TBLK, _FINAL), lambda i: (i, 0)),
        out_shape=jax.ShapeDtypeStruct((v, _FINAL), jnp.float32),
    )(tab_t)


def _sc_body(sample_t_hbm, id_tab_hbm, cat_tab_hbm, cmap_hbm, out_hbm,
             idx_all, tok2_v, cidx_v, idr_v, catr_v, sem_cmap, sem_rows, sem_out):
    wid = lax.axis_index("s") * _NC + lax.axis_index("c")
    col0 = wid * _COLS_W

    pltpu.sync_copy(sample_t_hbm.at[:, pl.ds(col0, _COLS_W)], idx_all)

    def out_slices(l):
        return (out_hbm.at[pl.ds(col0, _COLS_W), l, pl.ds(0, _ID_DIM)],
                out_hbm.at[pl.ds(col0, _COLS_W), l, pl.ds(_ID_DIM, _CAT_DIM)])

    def front(l, b, first):
        if first is None:
            _drain_out(l - _NBUF, b)
        else:
            @pl.when(first)
            def _():
                _drain_out(l - _NBUF, b)
        pltpu.async_copy(cmap_hbm.at[idx_all.at[l, :]],
                         cidx_v[b], sem_cmap[b])
        for t in range(_COLS_W // 16):
            sl = pl.ds(t * 16, 16)
            tok = idx_all[l, sl]
            tok2_v[b][sl] = tok + tok
        pltpu.async_copy(id_tab_hbm.at[tok2_v[b]],
                         idr_v[b], sem_rows[b])

    def _drain_out(l, b):
        oid, ocat = out_slices(l)
        pltpu.make_async_copy(oid, idr_v[b], sem_out[b]).wait()
        pltpu.make_async_copy(ocat, catr_v[b], sem_out[b]).wait()

    def mid(l, b):
        pltpu.make_async_copy(cmap_hbm.at[idx_all.at[l, :]],
                              cidx_v[b], sem_cmap[b]).wait()
        for t in range(_COLS_W // 16):
            sl = pl.ds(t * 16, 16)
            tok = idx_all[l, sl]
            cid = cidx_v[b][sl]
            cidx_v[b][sl] = jnp.where(tok == 0, jnp.zeros_like(cid),
                                      cid + cid)
        pltpu.async_copy(cat_tab_hbm.at[cidx_v[b]], catr_v[b], sem_rows[b])

    def back(l, b):
        pltpu.make_async_copy(id_tab_hbm.at[tok2_v[b]],
                              idr_v[b], sem_rows[b]).wait()
        pltpu.make_async_copy(cat_tab_hbm.at[cidx_v[b]],
                              catr_v[b], sem_rows[b]).wait()
        oid, ocat = out_slices(l)
        pltpu.async_copy(idr_v[b], oid, sem_out[b])
        pltpu.async_copy(catr_v[b], ocat, sem_out[b])

    def step(i, carry):
        for b in range(_NBUF):
            front(i * _NBUF + b, b, i >= 1)
        for b in range(_NBUF):
            mid(i * _NBUF + b, b)
        for b in range(_NBUF):
            back(i * _NBUF + b, b)
        return carry

    lax.fori_loop(0, _STEPS, step, 0)

    base = _STEPS * _NBUF
    for b in range(_REM):
        front(base + b, b, None)
    for b in range(_REM):
        mid(base + b, b)
    for b in range(_REM):
        back(base + b, b)

    for b in range(_REM, _NBUF):
        _drain_out(base - _NBUF + b, b)
    for b in range(_REM):
        _drain_out(base + b, b)


@jax.jit
def kernel(sample, item_id_table, category_table, category_map):
    mesh = plsc.VectorSubcoreMesh(core_axis_name="c", subcore_axis_name="s")
    f = functools.partial(
        pl.kernel,
        out_type=jax.ShapeDtypeStruct((_B, _L, _FINAL), jnp.float32),
        mesh=mesh,
        compiler_params=pltpu.CompilerParams(use_tc_tiling_on_sc=False),
        scratch_types=[
            pltpu.VMEM((_L, _COLS_W), jnp.int32),
            [pltpu.VMEM((_COLS_W,), jnp.int32)] * _NBUF,
            [pltpu.VMEM((_COLS_W,), jnp.int32)] * _NBUF,
            [pltpu.VMEM((_COLS_W, _ID_DIM), jnp.float32)] * _NBUF,
            [pltpu.VMEM((_COLS_W, _CAT_DIM), jnp.float32)] * _NBUF,
            [pltpu.SemaphoreType.DMA] * _NBUF,
            [pltpu.SemaphoreType.DMA] * _NBUF,
            [pltpu.SemaphoreType.DMA] * _NBUF,
        ],
    )(_sc_body)
    id2 = _transpose_table(item_id_table.T, 1000000).reshape(2000000, _ID_DIM)
    cat2 = _transpose_table(category_table.T, 100000).reshape(200000, _CAT_DIM)
    return f(sample.T, id2, cat2, category_map)


# speedup vs baseline: 3.1666x; 1.0359x over previous
import functools

import jax
import jax.numpy as jnp
from jax import lax
from jax.experimental import pallas as pl
from jax.experimental.pallas import tpu as pltpu
from jax.experimental.pallas import tpu_sc as plsc

_ID_DIM = 64
_CAT_DIM = 64
_FINAL = _ID_DIM + _CAT_DIM
_B = 4096
_L = 200

_NC = 2
_NS = 16
_NW = _NC * _NS
_COLS_W = _B // _NW
_NBUF = 6
_REM = _L % _NBUF
_STEPS = _L // _NBUF
_TBLK = 16384


def _transpose_body(src_ref, dst_ref):
    t = src_ref[...].T
    dst_ref[:, 0:64] = t
    dst_ref[:, 64:128] = jnp.zeros_like(t)


def _transpose_table(tab_t, v):
    grid = (v + _TBLK - 1) // _TBLK
    return pl.pallas_call(
        _transpose_body,
        grid=(grid,),
        in_specs=[pl.BlockSpec((64, _TBLK), lambda i: (0, i))],
        out_specs=pl.BlockSpec((_TBLK, _FINAL), lambda i: (i, 0)),
        out_shape=jax.ShapeDtypeStruct((v, _FINAL), jnp.float32),
    )(tab_t)


def _sc_body(sample_t_hbm, id_tab_hbm, cat_tab_hbm, cmap_hbm, out_hbm,
             idx_all, tok2_v, cidx_v, idr_v, catr_v, sem_cmap, sem_rows, sem_out):
    wid = lax.axis_index("s") * _NC + lax.axis_index("c")
    col0 = wid * _COLS_W

    pltpu.sync_copy(sample_t_hbm.at[:, pl.ds(col0, _COLS_W)], idx_all)

    def out_slices(l):
        return (out_hbm.at[pl.ds(col0, _COLS_W), l, pl.ds(0, _ID_DIM)],
                out_hbm.at[pl.ds(col0, _COLS_W), l, pl.ds(_ID_DIM, _CAT_DIM)])

    def front(l, b, first):
        if first is None:
            _drain_out(l - _NBUF, b)
        else:
            @pl.when(first)
            def _():
                _drain_out(l - _NBUF, b)
        pltpu.async_copy(cmap_hbm.at[idx_all.at[l, :]],
                         cidx_v[b], sem_cmap[b])
        for t in range(_COLS_W // 16):
            sl = pl.ds(t * 16, 16)
            tok = idx_all[l, sl]
            tok2_v[b][sl] = tok + tok
        pltpu.async_copy(id_tab_hbm.at[tok2_v[b]],
                         idr_v[b], sem_rows[b])

    def _drain_out(l, b):
        oid, ocat = out_slices(l)
        pltpu.make_async_copy(oid, idr_v[b], sem_out[b]).wait()
        pltpu.make_async_copy(ocat, catr_v[b], sem_out[b]).wait()

    def mid(l, b):
        pltpu.make_async_copy(cmap_hbm.at[idx_all.at[l, :]],
                              cidx_v[b], sem_cmap[b]).wait()
        for t in range(_COLS_W // 16):
            sl = pl.ds(t * 16, 16)
            tok = idx_all[l, sl]
            cid = cidx_v[b][sl]
            cidx_v[b][sl] = jnp.where(tok == 0, jnp.zeros_like(cid),
                                      cid + cid)
        pltpu.async_copy(cat_tab_hbm.at[cidx_v[b]], catr_v[b], sem_rows[b])

    def back(l, b):
        pltpu.make_async_copy(id_tab_hbm.at[tok2_v[b]],
                              idr_v[b], sem_rows[b]).wait()
        pltpu.make_async_copy(cat_tab_hbm.at[cidx_v[b]],
                              catr_v[b], sem_rows[b]).wait()
        oid, ocat = out_slices(l)
        pltpu.async_copy(idr_v[b], oid, sem_out[b])
        pltpu.async_copy(catr_v[b], ocat, sem_out[b])

    def step(i, carry):
        for b in range(_NBUF):
            front(i * _NBUF + b, b, i >= 1)
        for b in range(_NBUF):
            mid(i * _NBUF + b, b)
        for b in range(_NBUF):
            back(i * _NBUF + b, b)
        return carry

    lax.fori_loop(0, _STEPS, step, 0)

    base = _STEPS * _NBUF
    for b in range(_REM):
        front(base + b, b, None)
    for b in range(_REM):
        mid(base + b, b)
    for b in range(_REM):
        back(base + b, b)

    for b in range(_REM, _NBUF):
        _drain_out(base - _NBUF + b, b)
    for b in range(_REM):
        _drain_out(base + b, b)


@jax.jit
def kernel(sample, item_id_table, category_table, category_map):
    mesh = plsc.VectorSubcoreMesh(core_axis_name="c", subcore_axis_name="s")
    f = functools.partial(
        pl.kernel,
        out_type=jax.ShapeDtypeStruct((_B, _L, _FINAL), jnp.float32),
        mesh=mesh,
        compiler_params=pltpu.CompilerParams(use_tc_tiling_on_sc=False),
        scratch_types=[
            pltpu.VMEM((_L, _COLS_W), jnp.int32),
            [pltpu.VMEM((_COLS_W,), jnp.int32)] * _NBUF,
            [pltpu.VMEM((_COLS_W,), jnp.int32)] * _NBUF,
            [pltpu.VMEM((_COLS_W, _ID_DIM), jnp.float32)] * _NBUF,
            [pltpu.VMEM((_COLS_W, _CAT_DIM), jnp.float32)] * _NBUF,
            [pltpu.SemaphoreType.DMA] * _NBUF,
            [pltpu.SemaphoreType.DMA] * _NBUF,
            [pltpu.SemaphoreType.DMA] * _NBUF,
        ],
    )(_sc_body)
    id2 = _transpose_table(item_id_table.T, 1000000).reshape(2000000, _ID_DIM)
    cat2 = _transpose_table(category_table.T, 100000).reshape(200000, _CAT_DIM)
    return f(sample.T, id2, cat2, category_map)


# speedup vs baseline: 3.2024x; 1.0113x over previous
import functools

import jax
import jax.numpy as jnp
from jax import lax
from jax.experimental import pallas as pl
from jax.experimental.pallas import tpu as pltpu
from jax.experimental.pallas import tpu_sc as plsc

_ID_DIM = 64
_CAT_DIM = 64
_FINAL = _ID_DIM + _CAT_DIM
_B = 4096
_L = 200

_NC = 2
_NS = 16
_NW = _NC * _NS
_COLS_W = _B // _NW
_NBUF = 6
_REM = _L % _NBUF
_STEPS = _L // _NBUF
_TBLK = 32768


def _transpose_body(src_ref, dst_ref):
    t = src_ref[...].T
    dst_ref[:, 0:64] = t
    dst_ref[:, 64:128] = jnp.zeros_like(t)


def _transpose_table(tab_t, v):
    grid = (v + _TBLK - 1) // _TBLK
    return pl.pallas_call(
        _transpose_body,
        grid=(grid,),
        in_specs=[pl.BlockSpec((64, _TBLK), lambda i: (0, i))],
        out_specs=pl.BlockSpec((_TBLK, _FINAL), lambda i: (i, 0)),
        out_shape=jax.ShapeDtypeStruct((v, _FINAL), jnp.float32),
    )(tab_t)


def _sc_body(sample_t_hbm, id_tab_hbm, cat_tab_hbm, cmap_hbm, out_hbm,
             idx_all, tok2_v, cidx_v, idr_v, catr_v, sem_cmap, sem_rows, sem_out):
    wid = lax.axis_index("s") * _NC + lax.axis_index("c")
    col0 = wid * _COLS_W

    pltpu.sync_copy(sample_t_hbm.at[:, pl.ds(col0, _COLS_W)], idx_all)

    def out_slices(l):
        return (out_hbm.at[pl.ds(col0, _COLS_W), l, pl.ds(0, _ID_DIM)],
                out_hbm.at[pl.ds(col0, _COLS_W), l, pl.ds(_ID_DIM, _CAT_DIM)])

    def front(l, b, first):
        if first is None:
            _drain_out(l - _NBUF, b)
        else:
            @pl.when(first)
            def _():
                _drain_out(l - _NBUF, b)
        pltpu.async_copy(cmap_hbm.at[idx_all.at[l, :]],
                         cidx_v[b], sem_cmap[b])
        for t in range(_COLS_W // 16):
            sl = pl.ds(t * 16, 16)
            tok = idx_all[l, sl]
            tok2_v[b][sl] = tok + tok
        pltpu.async_copy(id_tab_hbm.at[tok2_v[b]],
                         idr_v[b], sem_rows[b])

    def _drain_out(l, b):
        oid, ocat = out_slices(l)
        pltpu.make_async_copy(oid, idr_v[b], sem_out[b]).wait()
        pltpu.make_async_copy(ocat, catr_v[b], sem_out[b]).wait()

    def mid(l, b):
        pltpu.make_async_copy(cmap_hbm.at[idx_all.at[l, :]],
                              cidx_v[b], sem_cmap[b]).wait()
        for t in range(_COLS_W // 16):
            sl = pl.ds(t * 16, 16)
            tok = idx_all[l, sl]
            cid = cidx_v[b][sl]
            cidx_v[b][sl] = jnp.where(tok == 0, jnp.zeros_like(cid),
                                      cid + cid)
        pltpu.async_copy(cat_tab_hbm.at[cidx_v[b]], catr_v[b], sem_rows[b])

    def back(l, b):
        pltpu.make_async_copy(id_tab_hbm.at[tok2_v[b]],
                              idr_v[b], sem_rows[b]).wait()
        pltpu.make_async_copy(cat_tab_hbm.at[cidx_v[b]],
                              catr_v[b], sem_rows[b]).wait()
        oid, ocat = out_slices(l)
        pltpu.async_copy(idr_v[b], oid, sem_out[b])
        pltpu.async_copy(catr_v[b], ocat, sem_out[b])

    def step(i, carry):
        for b in range(_NBUF):
            front(i * _NBUF + b, b, i >= 1)
        for b in range(_NBUF):
            mid(i * _NBUF + b, b)
        for b in range(_NBUF):
            back(i * _NBUF + b, b)
        return carry

    lax.fori_loop(0, _STEPS, step, 0)

    base = _STEPS * _NBUF
    for b in range(_REM):
        front(base + b, b, None)
    for b in range(_REM):
        mid(base + b, b)
    for b in range(_REM):
        back(base + b, b)

    for b in range(_REM, _NBUF):
        _drain_out(base - _NBUF + b, b)
    for b in range(_REM):
        _drain_out(base + b, b)


@jax.jit
def kernel(sample, item_id_table, category_table, category_map):
    mesh = plsc.VectorSubcoreMesh(core_axis_name="c", subcore_axis_name="s")
    f = functools.partial(
        pl.kernel,
        out_type=jax.ShapeDtypeStruct((_B, _L, _FINAL), jnp.float32),
        mesh=mesh,
        compiler_params=pltpu.CompilerParams(use_tc_tiling_on_sc=False),
        scratch_types=[
            pltpu.VMEM((_L, _COLS_W), jnp.int32),
            [pltpu.VMEM((_COLS_W,), jnp.int32)] * _NBUF,
            [pltpu.VMEM((_COLS_W,), jnp.int32)] * _NBUF,
            [pltpu.VMEM((_COLS_W, _ID_DIM), jnp.float32)] * _NBUF,
            [pltpu.VMEM((_COLS_W, _CAT_DIM), jnp.float32)] * _NBUF,
            [pltpu.SemaphoreType.DMA] * _NBUF,
            [pltpu.SemaphoreType.DMA] * _NBUF,
            [pltpu.SemaphoreType.DMA] * _NBUF,
        ],
    )(_sc_body)
    id2 = _transpose_table(item_id_table.T, 1000000).reshape(2000000, _ID_DIM)
    cat2 = _transpose_table(category_table.T, 100000).reshape(200000, _CAT_DIM)
    return f(sample.T, id2, cat2, category_map)
